# baseline (speedup 1.0000x reference)
import functools

import jax
import jax.numpy as jnp
from jax import lax
from jax.experimental import pallas as pl
from jax.experimental.pallas import tpu as pltpu
from jax.experimental.pallas import tpu_sc as plsc

N = 50000
E = 800000
CIN = 16
H = 64
FH = 32
NP = 51200
RT = NP // 16
DUMP = N

NB = 16
NG = 25
NCH = NB * NG
EW = NCH * 128
EP = EW * 16

BR = 2048
NBLK = NP // BR


def _sc_body(table, pks, out, pkbuf, idx_s, idx_d, rows, accum,
             sem_i, sem_g, sem_s):
    c = lax.axis_index("c")
    s = lax.axis_index("s")
    coff = c * NP
    ebase = s * EW

    def load_pk(g):
        b = lax.rem(g, 2)
        pltpu.async_copy(pks.at[pl.ds(ebase + g * NB * 128, NB * 128)],
                         pkbuf.at[b], sem_i)

    def wait_pk():
        pltpu.make_async_copy(pks.at[pl.ds(0, NB * 128)], pkbuf.at[0],
                              sem_i).wait()

    def unpack(g):
        b = lax.rem(g, 2)
        for i in range(NB):
            for j in range(8):
                v = pkbuf[b, pl.ds((i * 8 + j) * 16, 16)]
                idx_s[b, i, pl.ds(j * 16, 16)] = (v & 0xFFFF) + coff
                idx_d[b, i, pl.ds(j * 16, 16)] = (
                    lax.shift_right_logical(v, 16))

    def fire_gathers(g):
        b = lax.rem(g, 2)
        for i in range(NB):
            pltpu.async_copy(table.at[idx_s.at[b, i]], rows.at[b, i], sem_g)

    def wait_gathers():
        for i in range(NB):
            pltpu.make_async_copy(table.at[idx_s.at[0, i]], rows.at[0, i],
                                  sem_g).wait()

    def fire_scatters(g):
        b = lax.rem(g, 2)
        for i in range(NB):
            pltpu.async_copy(rows.at[b, i], accum.at[idx_d.at[b, i]], sem_s,
                             add=True)

    def wait_scatters():
        for i in range(NB):
            pltpu.make_async_copy(rows.at[0, i], accum.at[idx_d.at[0, i]],
                                  sem_s).wait()

    def zrow(r, carry):
        rows[0, 0, r, pl.ds(0, FH)] = jnp.zeros((FH,), jnp.bfloat16)
        return carry

    lax.fori_loop(0, 128, zrow, 0)
    for k in range(RT // 128):
        pltpu.sync_copy(rows.at[0, 0], accum.at[pl.ds(s * RT + k * 128, 128)])
    plsc.subcore_barrier()

    load_pk(0)
    wait_pk()
    unpack(0)
    fire_gathers(0)
    load_pk(1)

    def group(g, carry):
        @pl.when(g + 1 < NG)
        def _():
            wait_pk()

        @pl.when(g >= 1)
        def _():
            wait_scatters()

        @pl.when(g + 1 < NG)
        def _():
            unpack(g + 1)
            fire_gathers(g + 1)

        wait_gathers()
        fire_scatters(g)

        @pl.when(g + 2 < NG)
        def _():
            load_pk(g + 2)

        return carry

    lax.fori_loop(0, NG, group, 0)
    wait_scatters()
    plsc.subcore_barrier()

    pltpu.sync_copy(accum.at[pl.ds(s * RT, RT)],
                    out.at[c, pl.ds(s * RT, RT)])


_sc_scatter = functools.partial(
    pl.kernel,
    out_type=jax.ShapeDtypeStruct((2, NP, FH), jnp.bfloat16),
    mesh=plsc.VectorSubcoreMesh(core_axis_name="c", subcore_axis_name="s"),
    scratch_types=[
        pltpu.VMEM((2, NB * 128), jnp.int32),
        pltpu.VMEM((2, NB, 128), jnp.int32),
        pltpu.VMEM((2, NB, 128), jnp.int32),
        pltpu.VMEM((2, NB, 128, FH), jnp.bfloat16),
        pltpu.VMEM_SHARED((NP, FH), jnp.bfloat16),
        pltpu.SemaphoreType.DMA,
        pltpu.SemaphoreType.DMA,
        pltpu.SemaphoreType.DMA,
    ],
    compiler_params=pltpu.CompilerParams(use_tc_tiling_on_sc=False),
)(_sc_body)


def _lrelu(x):
    return jnp.where(x >= 0, x, 0.1 * x)


def _tc1_body(x_ref, wn_ref, bn_ref, out_ref):
    y = jnp.dot(x_ref[...], wn_ref[...], preferred_element_type=jnp.float32)
    y = (y + bn_ref[...]).astype(jnp.bfloat16)
    for hh in range(2):
        out_ref[hh] = y[:, hh * FH:(hh + 1) * FH]


def _tc2_body(x_ref, a_ref, dn_ref, ws_ref, bs_ref, wn2_ref, bn2_ref,
              x1_ref, n2_ref):
    agg = jnp.concatenate([a_ref[0], a_ref[1]], axis=-1).astype(jnp.float32)
    dn = jnp.maximum(dn_ref[...], 1.0)
    y = jnp.dot(x_ref[...], ws_ref[...], preferred_element_type=jnp.float32)
    x1 = _lrelu(y + bs_ref[...] + agg / dn)
    x1_ref[...] = x1
    n2 = jnp.dot(x1, wn2_ref[...], preferred_element_type=jnp.float32)
    n2 = (n2 + bn2_ref[...]).astype(jnp.bfloat16)
    for hh in range(2):
        n2_ref[hh] = n2[:, hh * FH:(hh + 1) * FH]


def _tc3_body(x1_ref, a_ref, dn_ref, ws_ref, bs_ref, wp1_ref, bp1_ref,
              wp2_ref, bp2_ref, out_ref, acc_ref):
    i = pl.program_id(0)
    agg = jnp.concatenate([a_ref[0], a_ref[1]], axis=-1).astype(jnp.float32)
    dn = jnp.maximum(dn_ref[...], 1.0)
    y = jnp.dot(x1_ref[...], ws_ref[...], preferred_element_type=jnp.float32)
    x2 = _lrelu(y + bs_ref[...] + agg / dn)
    rid = i * BR + lax.broadcasted_iota(jnp.int32, (BR, 1), 0)
    x2 = jnp.where(rid < N, x2, 0.0)

    @pl.when(i == 0)
    def _init():
        acc_ref[...] = jnp.zeros_like(acc_ref)

    acc_ref[...] += jnp.sum(x2, axis=0, keepdims=True)

    @pl.when(i == NBLK - 1)
    def _final():
        pooled = acc_ref[...] / float(N)
        h = jnp.dot(pooled, wp1_ref[...], preferred_element_type=jnp.float32)
        h = _lrelu(h + bp1_ref[...])
        o = jnp.dot(h, wp2_ref[...], preferred_element_type=jnp.float32)
        out_ref[...] = o + bp2_ref[...]


def _full(shape):
    nd = len(shape)
    return pl.BlockSpec(shape, lambda i: (0,) * nd)


_tc1 = pl.pallas_call(
    _tc1_body,
    grid=(NBLK,),
    in_specs=[
        pl.BlockSpec((BR, CIN), lambda i: (i, 0)),
        _full((CIN, H)),
        _full((1, H)),
    ],
    out_specs=pl.BlockSpec((2, BR, FH), lambda i: (0, i, 0)),
    out_shape=jax.ShapeDtypeStruct((2, NP, FH), jnp.bfloat16),
)

_tc2 = pl.pallas_call(
    _tc2_body,
    grid=(NBLK,),
    in_specs=[
        pl.BlockSpec((BR, CIN), lambda i: (i, 0)),
        pl.BlockSpec((2, BR, FH), lambda i: (0, i, 0)),
        pl.BlockSpec((BR, 1), lambda i: (i, 0)),
        _full((CIN, H)),
        _full((1, H)),
        _full((H, H)),
        _full((1, H)),
    ],
    out_specs=[
        pl.BlockSpec((BR, H), lambda i: (i, 0)),
        pl.BlockSpec((2, BR, FH), lambda i: (0, i, 0)),
    ],
    out_shape=[
        jax.ShapeDtypeStruct((NP, H), jnp.float32),
        jax.ShapeDtypeStruct((2, NP, FH), jnp.bfloat16),
    ],
)

_tc3 = pl.pallas_call(
    _tc3_body,
    grid=(NBLK,),
    in_specs=[
        pl.BlockSpec((BR, H), lambda i: (i, 0)),
        pl.BlockSpec((2, BR, FH), lambda i: (0, i, 0)),
        pl.BlockSpec((BR, 1), lambda i: (i, 0)),
        _full((H, H)),
        _full((1, H)),
        _full((H, H)),
        _full((1, H)),
        _full((H, 128)),
        _full((1, 128)),
    ],
    out_specs=pl.BlockSpec((1, 128), lambda i: (0, 0)),
    out_shape=jax.ShapeDtypeStruct((1, 128), jnp.float32),
    scratch_shapes=[pltpu.VMEM((1, H), jnp.float32)],
)


def kernel(node_feats, edge_index, deg, W_self1, b_self1, W_neigh1, b_neigh1,
           W_self2, b_self2, W_neigh2, b_neigh2, W_p1, b_p1, W_p2, b_p2):
    f32 = jnp.float32
    xp = node_feats.reshape(N, CIN)

    src = edge_index[0].astype(jnp.uint32)
    dst = edge_index[1].astype(jnp.uint32)
    pad = EP - E
    pk = jnp.concatenate([(dst << 16) | src,
                          jnp.full((pad,), jnp.uint32(DUMP) << 16)])
    pks = lax.bitcast_convert_type(pk, jnp.int32)

    dn = deg.astype(f32).reshape(N, 1)

    bs1 = b_self1.reshape(1, H).astype(f32)
    bn1 = b_neigh1.reshape(1, H).astype(f32)
    bs2 = b_self2.reshape(1, H).astype(f32)
    bn2 = b_neigh2.reshape(1, H).astype(f32)
    bp1 = b_p1.reshape(1, H).astype(f32)
    bp2 = b_p2.reshape(1, 128).astype(f32)

    n1 = _tc1(xp, W_neigh1.astype(f32), bn1)
    a1 = _sc_scatter(n1.reshape(2 * NP, FH), pks)
    x1, n2 = _tc2(xp, a1, dn, W_self1.astype(f32), bs1,
                  W_neigh2.astype(f32), bn2)
    a2 = _sc_scatter(n2.reshape(2 * NP, FH), pks)
    out = _tc3(x1, a2, dn, W_self2.astype(f32), bs2,
               W_p1.astype(f32), bp1, W_p2.astype(f32), bp2)
    return out


# speedup vs baseline: 1.0080x; 1.0080x over previous
import functools

import jax
import jax.numpy as jnp
from jax import lax
from jax.experimental import pallas as pl
from jax.experimental.pallas import tpu as pltpu
from jax.experimental.pallas import tpu_sc as plsc

N = 50000
E = 800000
CIN = 16
H = 64
FH = 32
NP = 51200
RT = NP // 16
DUMP = N

NB = 16
NG = 25
NCH = NB * NG
EW = NCH * 128
EP = EW * 16

BR = 2048
NBLK = NP // BR


def _sc_body(table, pks, out, pkbuf, idx_s, idx_d, rows, accum,
             sem_i, sem_g, sem_s):
    c = lax.axis_index("c")
    s = lax.axis_index("s")
    coff = c * NP
    ebase = s * EW

    def load_pk(g):
        b = lax.rem(g, 2)
        pltpu.async_copy(pks.at[pl.ds(ebase + g * NB * 128, NB * 128)],
                         pkbuf.at[b], sem_i)

    def wait_pk():
        pltpu.make_async_copy(pks.at[pl.ds(0, NB * 128)], pkbuf.at[0],
                              sem_i).wait()

    def unpack(g):
        b = lax.rem(g, 2)
        for i in range(NB):
            for j in range(8):
                v = pkbuf[b, pl.ds((i * 8 + j) * 16, 16)]
                idx_s[b, i, pl.ds(j * 16, 16)] = (v & 0xFFFF) + coff
                idx_d[b, i, pl.ds(j * 16, 16)] = (
                    lax.shift_right_logical(v, 16))

    def fire_gathers(g):
        b = lax.rem(g, 2)
        for i in range(NB):
            pltpu.async_copy(table.at[idx_s.at[b, i]], rows.at[b, i], sem_g)

    def wait_gathers():
        for i in range(NB):
            pltpu.make_async_copy(table.at[idx_s.at[0, i]], rows.at[0, i],
                                  sem_g).wait()

    def fire_scatters(g):
        b = lax.rem(g, 2)
        for i in range(NB):
            pltpu.async_copy(rows.at[b, i], accum.at[idx_d.at[b, i]], sem_s,
                             add=True)

    def wait_scatters():
        for i in range(NB):
            pltpu.make_async_copy(rows.at[0, i], accum.at[idx_d.at[0, i]],
                                  sem_s).wait()

    def zrow(r, carry):
        rows[0, 0, r, pl.ds(0, FH)] = jnp.zeros((FH,), jnp.bfloat16)
        return carry

    lax.fori_loop(0, 128, zrow, 0)
    for k in range(RT // 128):
        pltpu.sync_copy(rows.at[0, 0], accum.at[pl.ds(s * RT + k * 128, 128)])
    plsc.subcore_barrier()

    load_pk(0)
    wait_pk()
    unpack(0)
    fire_gathers(0)
    load_pk(1)

    def group(g, carry):
        @pl.when(g + 1 < NG)
        def _():
            wait_pk()

        @pl.when(g >= 1)
        def _():
            wait_scatters()

        @pl.when(g + 1 < NG)
        def _():
            unpack(g + 1)
            fire_gathers(g + 1)

        wait_gathers()
        fire_scatters(g)

        @pl.when(g + 2 < NG)
        def _():
            load_pk(g + 2)

        return carry

    lax.fori_loop(0, NG, group, 0)
    wait_scatters()
    plsc.subcore_barrier()

    pltpu.sync_copy(accum.at[pl.ds(s * RT, RT)],
                    out.at[c, pl.ds(s * RT, RT)])


_sc_scatter = functools.partial(
    pl.kernel,
    out_type=jax.ShapeDtypeStruct((2, NP, FH), jnp.bfloat16),
    mesh=plsc.VectorSubcoreMesh(core_axis_name="c", subcore_axis_name="s"),
    scratch_types=[
        pltpu.VMEM((2, NB * 128), jnp.int32),
        pltpu.VMEM((2, NB, 128), jnp.int32),
        pltpu.VMEM((2, NB, 128), jnp.int32),
        pltpu.VMEM((2, NB, 128, FH), jnp.bfloat16),
        pltpu.VMEM_SHARED((NP, FH), jnp.bfloat16),
        pltpu.SemaphoreType.DMA,
        pltpu.SemaphoreType.DMA,
        pltpu.SemaphoreType.DMA,
    ],
    compiler_params=pltpu.CompilerParams(use_tc_tiling_on_sc=False),
)(_sc_body)


def _lrelu(x):
    return jnp.where(x >= 0, x, 0.1 * x)


def _tc1_body(x_ref, wn_ref, bn_ref, out_ref):
    y = jnp.dot(x_ref[...], wn_ref[...], preferred_element_type=jnp.float32)
    y = (y + bn_ref[...]).astype(jnp.bfloat16)
    for hh in range(2):
        out_ref[hh] = y[:, hh * FH:(hh + 1) * FH]


def _tc2_body(x_ref, a_ref, dn_ref, ws_ref, bs_ref, wn2_ref, bn2_ref,
              x1_ref, n2_ref):
    agg = jnp.concatenate([a_ref[0], a_ref[1]], axis=-1).astype(jnp.float32)
    dn = jnp.maximum(dn_ref[...], 1.0)
    y = jnp.dot(x_ref[...], ws_ref[...], preferred_element_type=jnp.float32)
    x1 = _lrelu(y + bs_ref[...] + agg / dn)
    x1_ref[...] = x1
    n2 = jnp.dot(x1, wn2_ref[...], preferred_element_type=jnp.float32)
    n2 = (n2 + bn2_ref[...]).astype(jnp.bfloat16)
    for hh in range(2):
        n2_ref[hh] = n2[:, hh * FH:(hh + 1) * FH]


def _tc3_body(x1_ref, a_ref, dn_ref, ws_ref, bs_ref, wp1_ref, bp1_ref,
              wp2_ref, bp2_ref, out_ref, acc_ref):
    i = pl.program_id(0)
    agg = jnp.concatenate([a_ref[0], a_ref[1]], axis=-1).astype(jnp.float32)
    dn = jnp.maximum(dn_ref[...], 1.0)
    y = jnp.dot(x1_ref[...], ws_ref[...], preferred_element_type=jnp.float32)
    x2 = _lrelu(y + bs_ref[...] + agg / dn)
    rid = i * BR + lax.broadcasted_iota(jnp.int32, (BR, 1), 0)
    x2 = jnp.where(rid < N, x2, 0.0)

    @pl.when(i == 0)
    def _init():
        acc_ref[...] = jnp.zeros_like(acc_ref)

    acc_ref[...] += jnp.sum(x2, axis=0, keepdims=True)

    @pl.when(i == NBLK - 1)
    def _final():
        pooled = acc_ref[...] / float(N)
        h = jnp.dot(pooled, wp1_ref[...], preferred_element_type=jnp.float32)
        h = _lrelu(h + bp1_ref[...])
        o = jnp.dot(h, wp2_ref[...], preferred_element_type=jnp.float32)
        out_ref[...] = o + bp2_ref[...]


def _full(shape):
    nd = len(shape)
    return pl.BlockSpec(shape, lambda i: (0,) * nd)


_tc1 = pl.pallas_call(
    _tc1_body,
    grid=(NBLK,),
    in_specs=[
        pl.BlockSpec((BR, CIN), lambda i: (i, 0)),
        _full((CIN, H)),
        _full((1, H)),
    ],
    out_specs=pl.BlockSpec((2, BR, FH), lambda i: (0, i, 0)),
    out_shape=jax.ShapeDtypeStruct((2, NP, FH), jnp.bfloat16),
)

_tc2 = pl.pallas_call(
    _tc2_body,
    grid=(NBLK,),
    in_specs=[
        pl.BlockSpec((BR, CIN), lambda i: (i, 0)),
        pl.BlockSpec((2, BR, FH), lambda i: (0, i, 0)),
        pl.BlockSpec((BR, 1), lambda i: (i, 0)),
        _full((CIN, H)),
        _full((1, H)),
        _full((H, H)),
        _full((1, H)),
    ],
    out_specs=[
        pl.BlockSpec((BR, H), lambda i: (i, 0)),
        pl.BlockSpec((2, BR, FH), lambda i: (0, i, 0)),
    ],
    out_shape=[
        jax.ShapeDtypeStruct((NP, H), jnp.float32),
        jax.ShapeDtypeStruct((2, NP, FH), jnp.bfloat16),
    ],
)

_tc3 = pl.pallas_call(
    _tc3_body,
    grid=(NBLK,),
    in_specs=[
        pl.BlockSpec((BR, H), lambda i: (i, 0)),
        pl.BlockSpec((2, BR, FH), lambda i: (0, i, 0)),
        pl.BlockSpec((BR, 1), lambda i: (i, 0)),
        _full((H, H)),
        _full((1, H)),
        _full((H, H)),
        _full((1, H)),
        _full((H, 128)),
        _full((1, 128)),
    ],
    out_specs=pl.BlockSpec((1, 128), lambda i: (0, 0)),
    out_shape=jax.ShapeDtypeStruct((1, 128), jnp.float32),
    scratch_shapes=[pltpu.VMEM((1, H), jnp.float32)],
)


def kernel(node_feats, edge_index, deg, W_self1, b_self1, W_neigh1, b_neigh1,
           W_self2, b_self2, W_neigh2, b_neigh2, W_p1, b_p1, W_p2, b_p2):
    f32 = jnp.float32
    x = node_feats.reshape(N, CIN)
    xp = jnp.pad(x, ((0, NP - N), (0, 0)))

    src = edge_index[0].astype(jnp.uint32)
    dst = edge_index[1].astype(jnp.uint32)
    pad = EP - E
    pk = jnp.concatenate([(dst << 16) | src,
                          jnp.full((pad,), jnp.uint32(DUMP) << 16)])
    pks = lax.bitcast_convert_type(pk, jnp.int32)

    degf = jnp.concatenate([deg.astype(f32), jnp.ones((NP - N,), f32)])
    dn = degf.reshape(NP, 1)

    bs1 = b_self1.reshape(1, H).astype(f32)
    bn1 = b_neigh1.reshape(1, H).astype(f32)
    bs2 = b_self2.reshape(1, H).astype(f32)
    bn2 = b_neigh2.reshape(1, H).astype(f32)
    bp1 = b_p1.reshape(1, H).astype(f32)
    bp2 = b_p2.reshape(1, 128).astype(f32)

    n1 = _tc1(xp, W_neigh1.astype(f32), bn1)
    a1 = _sc_scatter(n1.reshape(2 * NP, FH), pks)
    x1, n2 = _tc2(xp, a1, dn, W_self1.astype(f32), bs1,
                  W_neigh2.astype(f32), bn2)
    a2 = _sc_scatter(n2.reshape(2 * NP, FH), pks)
    out = _tc3(x1, a2, dn, W_self2.astype(f32), bs2,
               W_p1.astype(f32), bp1, W_p2.astype(f32), bp2)
    return out


# speedup vs baseline: 1.0086x; 1.0005x over previous
import functools

import jax
import jax.numpy as jnp
from jax import lax
from jax.experimental import pallas as pl
from jax.experimental.pallas import tpu as pltpu
from jax.experimental.pallas import tpu_sc as plsc

N = 50000
E = 800000
CIN = 16
H = 64
FH = 32
NP = 51200
RT = NP // 16
DUMP = N

NB = 8
NG = 25
NCH = NB * NG
EW = NCH * 256
EP = EW * 16

BR = 2048
NBLK = NP // BR


def _sc_body(table, pks, out, pkbuf, idx_s, idx_d, rows, accum,
             sem_i, sem_g, sem_s):
    c = lax.axis_index("c")
    s = lax.axis_index("s")
    coff = c * NP
    ebase = s * EW

    def load_pk(g):
        b = lax.rem(g, 2)
        pltpu.async_copy(pks.at[pl.ds(ebase + g * NB * 256, NB * 256)],
                         pkbuf.at[b], sem_i)

    def wait_pk():
        pltpu.make_async_copy(pks.at[pl.ds(0, NB * 256)], pkbuf.at[0],
                              sem_i).wait()

    def unpack(g):
        b = lax.rem(g, 2)
        for i in range(NB):
            for j in range(16):
                v = pkbuf[b, pl.ds((i * 16 + j) * 16, 16)]
                idx_s[b, i, pl.ds(j * 16, 16)] = (v & 0xFFFF) + coff
                idx_d[b, i, pl.ds(j * 16, 16)] = (
                    lax.shift_right_logical(v, 16))

    def fire_gathers(g):
        b = lax.rem(g, 2)
        for i in range(NB):
            pltpu.async_copy(table.at[idx_s.at[b, i]], rows.at[b, i], sem_g)

    def wait_gathers():
        for i in range(NB):
            pltpu.make_async_copy(table.at[idx_s.at[0, i]], rows.at[0, i],
                                  sem_g).wait()

    def fire_scatters(g):
        b = lax.rem(g, 2)
        for i in range(NB):
            pltpu.async_copy(rows.at[b, i], accum.at[idx_d.at[b, i]], sem_s,
                             add=True)

    def wait_scatters():
        for i in range(NB):
            pltpu.make_async_copy(rows.at[0, i], accum.at[idx_d.at[0, i]],
                                  sem_s).wait()

    def zrow(r, carry):
        rows[0, 0, r, pl.ds(0, FH)] = jnp.zeros((FH,), jnp.bfloat16)
        return carry

    lax.fori_loop(0, 128, zrow, 0)
    for k in range(RT // 128):
        pltpu.sync_copy(rows.at[0, 0, pl.ds(0, 128)],
                        accum.at[pl.ds(s * RT + k * 128, 128)])
    plsc.subcore_barrier()

    load_pk(0)
    wait_pk()
    unpack(0)
    fire_gathers(0)
    load_pk(1)

    def group(g, carry):
        @pl.when(g + 1 < NG)
        def _():
            wait_pk()

        @pl.when(g >= 1)
        def _():
            wait_scatters()

        @pl.when(g + 1 < NG)
        def _():
            unpack(g + 1)
            fire_gathers(g + 1)

        wait_gathers()
        fire_scatters(g)

        @pl.when(g + 2 < NG)
        def _():
            load_pk(g + 2)

        return carry

    lax.fori_loop(0, NG, group, 0)
    wait_scatters()
    plsc.subcore_barrier()

    pltpu.sync_copy(accum.at[pl.ds(s * RT, RT)],
                    out.at[c, pl.ds(s * RT, RT)])


_sc_scatter = functools.partial(
    pl.kernel,
    out_type=jax.ShapeDtypeStruct((2, NP, FH), jnp.bfloat16),
    mesh=plsc.VectorSubcoreMesh(core_axis_name="c", subcore_axis_name="s"),
    scratch_types=[
        pltpu.VMEM((2, NB * 256), jnp.int32),
        pltpu.VMEM((2, NB, 256), jnp.int32),
        pltpu.VMEM((2, NB, 256), jnp.int32),
        pltpu.VMEM((2, NB, 256, FH), jnp.bfloat16),
        pltpu.VMEM_SHARED((NP, FH), jnp.bfloat16),
        pltpu.SemaphoreType.DMA,
        pltpu.SemaphoreType.DMA,
        pltpu.SemaphoreType.DMA,
    ],
    compiler_params=pltpu.CompilerParams(use_tc_tiling_on_sc=False),
)(_sc_body)


def _lrelu(x):
    return jnp.where(x >= 0, x, 0.1 * x)


def _tc1_body(x_ref, wn_ref, bn_ref, out_ref):
    y = jnp.dot(x_ref[...], wn_ref[...], preferred_element_type=jnp.float32)
    y = (y + bn_ref[...]).astype(jnp.bfloat16)
    for hh in range(2):
        out_ref[hh] = y[:, hh * FH:(hh + 1) * FH]


def _tc2_body(x_ref, a_ref, dn_ref, ws_ref, bs_ref, wn2_ref, bn2_ref,
              x1_ref, n2_ref):
    agg = jnp.concatenate([a_ref[0], a_ref[1]], axis=-1).astype(jnp.float32)
    dn = jnp.maximum(dn_ref[...], 1.0)
    y = jnp.dot(x_ref[...], ws_ref[...], preferred_element_type=jnp.float32)
    x1 = _lrelu(y + bs_ref[...] + agg / dn)
    x1_ref[...] = x1
    n2 = jnp.dot(x1, wn2_ref[...], preferred_element_type=jnp.float32)
    n2 = (n2 + bn2_ref[...]).astype(jnp.bfloat16)
    for hh in range(2):
        n2_ref[hh] = n2[:, hh * FH:(hh + 1) * FH]


def _tc3_body(x1_ref, a_ref, dn_ref, ws_ref, bs_ref, wp1_ref, bp1_ref,
              wp2_ref, bp2_ref, out_ref, acc_ref):
    i = pl.program_id(0)
    agg = jnp.concatenate([a_ref[0], a_ref[1]], axis=-1).astype(jnp.float32)
    dn = jnp.maximum(dn_ref[...], 1.0)
    y = jnp.dot(x1_ref[...], ws_ref[...], preferred_element_type=jnp.float32)
    x2 = _lrelu(y + bs_ref[...] + agg / dn)
    rid = i * BR + lax.broadcasted_iota(jnp.int32, (BR, 1), 0)
    x2 = jnp.where(rid < N, x2, 0.0)

    @pl.when(i == 0)
    def _init():
        acc_ref[...] = jnp.zeros_like(acc_ref)

    acc_ref[...] += jnp.sum(x2, axis=0, keepdims=True)

    @pl.when(i == NBLK - 1)
    def _final():
        pooled = acc_ref[...] / float(N)
        h = jnp.dot(pooled, wp1_ref[...], preferred_element_type=jnp.float32)
        h = _lrelu(h + bp1_ref[...])
        o = jnp.dot(h, wp2_ref[...], preferred_element_type=jnp.float32)
        out_ref[...] = o + bp2_ref[...]


def _full(shape):
    nd = len(shape)
    return pl.BlockSpec(shape, lambda i: (0,) * nd)


_tc1 = pl.pallas_call(
    _tc1_body,
    grid=(NBLK,),
    in_specs=[
        pl.BlockSpec((BR, CIN), lambda i: (i, 0)),
        _full((CIN, H)),
        _full((1, H)),
    ],
    out_specs=pl.BlockSpec((2, BR, FH), lambda i: (0, i, 0)),
    out_shape=jax.ShapeDtypeStruct((2, NP, FH), jnp.bfloat16),
)

_tc2 = pl.pallas_call(
    _tc2_body,
    grid=(NBLK,),
    in_specs=[
        pl.BlockSpec((BR, CIN), lambda i: (i, 0)),
        pl.BlockSpec((2, BR, FH), lambda i: (0, i, 0)),
        pl.BlockSpec((BR, 1), lambda i: (i, 0)),
        _full((CIN, H)),
        _full((1, H)),
        _full((H, H)),
        _full((1, H)),
    ],
    out_specs=[
        pl.BlockSpec((BR, H), lambda i: (i, 0)),
        pl.BlockSpec((2, BR, FH), lambda i: (0, i, 0)),
    ],
    out_shape=[
        jax.ShapeDtypeStruct((NP, H), jnp.float32),
        jax.ShapeDtypeStruct((2, NP, FH), jnp.bfloat16),
    ],
)

_tc3 = pl.pallas_call(
    _tc3_body,
    grid=(NBLK,),
    in_specs=[
        pl.BlockSpec((BR, H), lambda i: (i, 0)),
        pl.BlockSpec((2, BR, FH), lambda i: (0, i, 0)),
        pl.BlockSpec((BR, 1), lambda i: (i, 0)),
        _full((H, H)),
        _full((1, H)),
        _full((H, H)),
        _full((1, H)),
        _full((H, 128)),
        _full((1, 128)),
    ],
    out_specs=pl.BlockSpec((1, 128), lambda i: (0, 0)),
    out_shape=jax.ShapeDtypeStruct((1, 128), jnp.float32),
    scratch_shapes=[pltpu.VMEM((1, H), jnp.float32)],
)


def kernel(node_feats, edge_index, deg, W_self1, b_self1, W_neigh1, b_neigh1,
           W_self2, b_self2, W_neigh2, b_neigh2, W_p1, b_p1, W_p2, b_p2):
    f32 = jnp.float32
    x = node_feats.reshape(N, CIN)
    xp = jnp.pad(x, ((0, NP - N), (0, 0)))

    src = edge_index[0].astype(jnp.uint32)
    dst = edge_index[1].astype(jnp.uint32)
    pad = EP - E
    pk = jnp.concatenate([(dst << 16) | src,
                          jnp.full((pad,), jnp.uint32(DUMP) << 16)])
    pks = lax.bitcast_convert_type(pk, jnp.int32)

    degf = jnp.concatenate([deg.astype(f32), jnp.ones((NP - N,), f32)])
    dn = degf.reshape(NP, 1)

    bs1 = b_self1.reshape(1, H).astype(f32)
    bn1 = b_neigh1.reshape(1, H).astype(f32)
    bs2 = b_self2.reshape(1, H).astype(f32)
    bn2 = b_neigh2.reshape(1, H).astype(f32)
    bp1 = b_p1.reshape(1, H).astype(f32)
    bp2 = b_p2.reshape(1, 128).astype(f32)

    n1 = _tc1(xp, W_neigh1.astype(f32), bn1)
    a1 = _sc_scatter(n1.reshape(2 * NP, FH), pks)
    x1, n2 = _tc2(xp, a1, dn, W_self1.astype(f32), bs1,
                  W_neigh2.astype(f32), bn2)
    a2 = _sc_scatter(n2.reshape(2 * NP, FH), pks)
    out = _tc3(x1, a2, dn, W_self2.astype(f32), bs2,
               W_p1.astype(f32), bp1, W_p2.astype(f32), bp2)
    return out
